# baseline (speedup 1.0000x reference)
import dataclasses
import functools

import jax
import jax.numpy as jnp
from jax import lax
from jax.experimental import pallas as pl
from jax.experimental.pallas import tpu as pltpu
from jax.experimental.pallas import tpu_sc as plsc

N_SC_CORES = 2
N_SUBCORES = 16
N_WORKERS = N_SC_CORES * N_SUBCORES
CHUNK = 128


def _make_sc_agg(feat_dim, n_chunks, n_acc, with_deg, hist_rows=0,
                 blocks_split=None):
    del blocks_split
    idx_blk = 8
    n_iter = n_chunks // (idx_blk * N_WORKERS)
    rows_per_sub = n_acc // N_SUBCORES
    n_zero = rows_per_sub // CHUNK
    n_hist = hist_rows
    mesh = plsc.VectorSubcoreMesh(core_axis_name="c", subcore_axis_name="s")

    out_type = [jax.ShapeDtypeStruct((N_SC_CORES, n_acc, feat_dim), jnp.float32)]
    scratch = [
        pltpu.VMEM((idx_blk, CHUNK), jnp.int32),
        pltpu.VMEM((idx_blk, CHUNK), jnp.int32),
        pltpu.VMEM((CHUNK, feat_dim), jnp.float32),
        pltpu.VMEM((CHUNK, feat_dim), jnp.float32),
        pltpu.VMEM_SHARED((n_acc, feat_dim), jnp.float32),
        pltpu.SemaphoreType.DMA,
        pltpu.SemaphoreType.DMA,
    ]
    if with_deg:
        out_type.append(jax.ShapeDtypeStruct((N_WORKERS * n_hist,), jnp.float32))
        scratch.append(pltpu.VMEM((n_hist,), jnp.float32))

    cp = pltpu.CompilerParams()
    if "needs_layout_passes" in pltpu.CompilerParams.__dataclass_fields__:
        cp = dataclasses.replace(cp, needs_layout_passes=False)

    @functools.partial(pl.kernel, mesh=mesh, out_type=out_type,
                       scratch_types=scratch, compiler_params=cp)
    def sc_agg(*refs):
        if with_deg:
            (feat_hbm, srcs_hbm, dsts_hbm, zf_hbm,
             part_hbm, degh_hbm,
             src_v, dst_v, rows_a, rows_b, acc_sh, sem_a, sem_b, hist_v) = refs
        else:
            (feat_hbm, srcs_hbm, dsts_hbm, zf_hbm,
             part_hbm,
             src_v, dst_v, rows_a, rows_b, acc_sh, sem_a, sem_b) = refs
        c = lax.axis_index("c")
        s = lax.axis_index("s")
        w = c * N_SUBCORES + s
        row0 = s * rows_per_sub
        bufs = (rows_a, rows_b)
        sems = (sem_a, sem_b)

        with jax.named_scope("zero_acc"):
            pltpu.sync_copy(zf_hbm, rows_a)

            @pl.loop(0, n_zero)
            def _(k):
                pltpu.sync_copy(rows_a,
                                acc_sh.at[pl.ds(row0 + k * CHUNK, CHUNK)])

        if with_deg:
            zeros16 = jnp.zeros((16,), jnp.float32)

            @pl.loop(0, n_hist // 16)
            def _(i):
                hist_v[pl.ds(i * 16, 16)] = zeros16

        plsc.subcore_barrier()

        if with_deg:
            ones16 = jnp.full((16,), 1.0, jnp.float32)

        def run_edges(n_blocks):
            @pl.loop(0, n_blocks)
            def _(b):
                blk = (b * N_WORKERS + w) * idx_blk
                pltpu.sync_copy(srcs_hbm.at[pl.ds(blk, idx_blk)], src_v)
                pltpu.sync_copy(dsts_hbm.at[pl.ds(blk, idx_blk)], dst_v)

                copies = [None] * idx_blk
                copies[0] = pltpu.async_copy(
                    feat_hbm.at[src_v.at[0]], bufs[0], sems[0])
                for j in range(idx_blk):
                    if j + 1 < idx_blk:
                        copies[j + 1] = pltpu.async_copy(
                            feat_hbm.at[src_v.at[j + 1]],
                            bufs[(j + 1) % 2], sems[(j + 1) % 2])
                    if with_deg:
                        for k in range(CHUNK // 16):
                            idx = dst_v[j, pl.ds(k * 16, 16)]
                            plsc.addupdate_scatter(hist_v, [idx], ones16)
                    copies[j].wait()
                    pltpu.sync_copy(bufs[j % 2], acc_sh.at[dst_v.at[j]],
                                    add=True)

        with jax.named_scope("edges"):
            run_edges(n_iter)

        plsc.subcore_barrier()

        with jax.named_scope("writeout"):
            pltpu.sync_copy(acc_sh.at[pl.ds(row0, rows_per_sub)],
                            part_hbm.at[c].at[pl.ds(row0, rows_per_sub)])
            if with_deg:
                pltpu.sync_copy(hist_v, degh_hbm.at[pl.ds(w * n_hist, n_hist)])

    return sc_agg


def _tc_layer1(x, part, degp, W1l, b1, W1r, W2l, W2r, b2):
    n, f_in = x.shape
    h_dim = W1l.shape[0]
    c_dim = W2l.shape[0]

    def body(x_ref, p_ref, d_ref, w1l_ref, b1_ref, w1r_ref, w2l_ref,
             w2r_ref, b2_ref, h2l_ref, h2r_ref):
        psum = p_ref[0, :n, :] + p_ref[1, :n, :]
        deg = lax.dot_general(d_ref[...], jnp.ones((N_WORKERS, 1), jnp.float32),
                              (((0,), (0,)), ((), ())),
                              preferred_element_type=jnp.float32)[:n, :]
        inv = 1.0 / jnp.maximum(deg, 1.0)
        agg = psum * inv
        dn = (((1,), (1,)), ((), ()))
        h = jnp.maximum(
            lax.dot_general(agg, w1l_ref[...], dn,
                            preferred_element_type=jnp.float32)
            + b1_ref[...]
            + lax.dot_general(x_ref[...], w1r_ref[...], dn,
                              preferred_element_type=jnp.float32),
            0.0)
        h2l_ref[...] = h
        h2r_ref[...] = lax.dot_general(h, w2r_ref[...], dn,
                                       preferred_element_type=jnp.float32) + b2_ref[...]

    return pl.pallas_call(
        body,
        out_shape=[jax.ShapeDtypeStruct((n, f_in), jnp.float32),
                   jax.ShapeDtypeStruct((n, c_dim), jnp.float32)],
    )(x, part, degp, W1l, b1.reshape(1, h_dim), W1r, W2l, W2r,
      b2.reshape(1, c_dim))


def _tc_layer2(part2, degp, W2l, h2r, gamma, beta):
    n, c_dim = h2r.shape

    def body(q_ref, d_ref, w2l_ref, h2r_ref, g_ref, b_ref, o_ref):
        qsum = q_ref[0, :n, :] + q_ref[1, :n, :]
        deg = lax.dot_general(d_ref[...], jnp.ones((N_WORKERS, 1), jnp.float32),
                              (((0,), (0,)), ((), ())),
                              preferred_element_type=jnp.float32)[:n, :]
        inv = 1.0 / jnp.maximum(deg, 1.0)
        agg = qsum * inv
        dn = (((1,), (1,)), ((), ()))
        pre = lax.dot_general(agg, w2l_ref[...], dn,
                              preferred_element_type=jnp.float32) + h2r_ref[...]
        mean = jnp.mean(pre, axis=0, keepdims=True)
        cent = pre - mean
        var = jnp.mean(cent * cent, axis=0, keepdims=True)
        o_ref[...] = cent * lax.rsqrt(var + 1e-5) * g_ref[...] + b_ref[...]

    return pl.pallas_call(
        body,
        out_shape=jax.ShapeDtypeStruct((n, c_dim), jnp.float32),
    )(part2, degp, W2l, h2r, gamma.reshape(1, c_dim), beta.reshape(1, c_dim))


def kernel(x, edge_index, W1l, b1, W1r, W2l, b2, W2r, gamma, beta):
    n, f_in = x.shape
    e = edge_index.shape[1]
    c_dim = W2l.shape[0]

    span = N_WORKERS * CHUNK * 8
    e_pad = ((e + span - 1) // span) * span
    n_chunks = e_pad // CHUNK
    n_acc = ((n + 1 + N_SUBCORES * CHUNK - 1)
             // (N_SUBCORES * CHUNK)) * (N_SUBCORES * CHUNK)

    pad = e_pad - e
    src = jnp.concatenate(
        [edge_index[0], jnp.zeros((pad,), jnp.int32)]).reshape(n_chunks, CHUNK)
    pad_dst = n + (jnp.arange(pad, dtype=jnp.int32) % 16)
    dst = jnp.concatenate(
        [edge_index[1], pad_dst]).reshape(n_chunks, CHUNK)

    zf1 = jnp.zeros((CHUNK, f_in), jnp.float32)
    n_hist = ((n + 1 + 15) // 16) * 16

    sc1 = _make_sc_agg(f_in, n_chunks, n_acc, with_deg=True, hist_rows=n_hist)
    part1, degh = sc1(x, src, dst, zf1)
    degp = degh.reshape(N_WORKERS, n_hist)

    h, h2r = _tc_layer1(x, part1, degp, W1l, b1, W1r, W2l, W2r, b2)

    sc2 = _make_sc_agg(f_in, n_chunks, n_acc, with_deg=False)
    (part2,) = sc2(h, src, dst, zf1)

    return _tc_layer2(part2, degp, W2l, h2r, gamma, beta)


# speedup vs baseline: 1.0127x; 1.0127x over previous
import dataclasses
import functools

import jax
import jax.numpy as jnp
from jax import lax
from jax.experimental import pallas as pl
from jax.experimental.pallas import tpu as pltpu
from jax.experimental.pallas import tpu_sc as plsc

N_SC_CORES = 2
N_SUBCORES = 16
N_WORKERS = N_SC_CORES * N_SUBCORES
CHUNK = 128


def _make_sc_agg(feat_dim, n_chunks, n_acc, with_deg, hist_rows=0,
                 slow_core=1, slow_blocks=4):
    idx_blk = 8
    total_blocks = n_chunks // idx_blk
    w_s = slow_blocks
    w_f = (total_blocks - 8 * w_s) // 24
    assert 24 * w_f + 8 * w_s == total_blocks, (w_f, w_s, total_blocks)
    rows_per_sub = n_acc // N_SUBCORES
    n_zero = rows_per_sub // CHUNK
    n_hist = hist_rows
    mesh = plsc.VectorSubcoreMesh(core_axis_name="c", subcore_axis_name="s")

    out_type = [jax.ShapeDtypeStruct((N_SC_CORES, n_acc, feat_dim), jnp.float32)]
    scratch = [
        pltpu.VMEM((idx_blk, CHUNK), jnp.int32),
        pltpu.VMEM((idx_blk, CHUNK), jnp.int32),
        pltpu.VMEM((CHUNK, feat_dim), jnp.float32),
        pltpu.VMEM((CHUNK, feat_dim), jnp.float32),
        pltpu.VMEM_SHARED((n_acc, feat_dim), jnp.float32),
        pltpu.SemaphoreType.DMA,
        pltpu.SemaphoreType.DMA,
    ]
    if with_deg:
        out_type.append(jax.ShapeDtypeStruct((N_WORKERS * n_hist,), jnp.float32))
        scratch.append(pltpu.VMEM((n_hist,), jnp.float32))

    cp = pltpu.CompilerParams()
    if "needs_layout_passes" in pltpu.CompilerParams.__dataclass_fields__:
        cp = dataclasses.replace(cp, needs_layout_passes=False)

    @functools.partial(pl.kernel, mesh=mesh, out_type=out_type,
                       scratch_types=scratch, compiler_params=cp)
    def sc_agg(*refs):
        if with_deg:
            (feat_hbm, srcs_hbm, dsts_hbm, zf_hbm,
             part_hbm, degh_hbm,
             src_v, dst_v, rows_a, rows_b, acc_sh, sem_a, sem_b, hist_v) = refs
        else:
            (feat_hbm, srcs_hbm, dsts_hbm, zf_hbm,
             part_hbm,
             src_v, dst_v, rows_a, rows_b, acc_sh, sem_a, sem_b) = refs
        c = lax.axis_index("c")
        s = lax.axis_index("s")
        w = c * N_SUBCORES + s
        row0 = s * rows_per_sub
        bufs = (rows_a, rows_b)
        sems = (sem_a, sem_b)

        with jax.named_scope("zero_acc"):
            pltpu.sync_copy(zf_hbm, rows_a)

            @pl.loop(0, n_zero)
            def _(k):
                pltpu.sync_copy(rows_a,
                                acc_sh.at[pl.ds(row0 + k * CHUNK, CHUNK)])

        if with_deg:
            zeros16 = jnp.zeros((16,), jnp.float32)

            @pl.loop(0, n_hist // 16)
            def _(i):
                hist_v[pl.ds(i * 16, 16)] = zeros16

        plsc.subcore_barrier()

        if with_deg:
            ones16 = jnp.full((16,), 1.0, jnp.float32)

        def run_edges(n_blocks, start_block):
            @pl.loop(0, n_blocks)
            def _(b):
                blk = (start_block + b) * idx_blk
                pltpu.sync_copy(srcs_hbm.at[pl.ds(blk, idx_blk)], src_v)
                pltpu.sync_copy(dsts_hbm.at[pl.ds(blk, idx_blk)], dst_v)

                copies = [None] * idx_blk
                copies[0] = pltpu.async_copy(
                    feat_hbm.at[src_v.at[0]], bufs[0], sems[0])
                for j in range(idx_blk):
                    if j + 1 < idx_blk:
                        copies[j + 1] = pltpu.async_copy(
                            feat_hbm.at[src_v.at[j + 1]],
                            bufs[(j + 1) % 2], sems[(j + 1) % 2])
                    if with_deg:
                        for k in range(CHUNK // 16):
                            idx = dst_v[j, pl.ds(k * 16, 16)]
                            plsc.addupdate_scatter(hist_v, [idx], ones16)
                    copies[j].wait()
                    pltpu.sync_copy(bufs[j % 2], acc_sh.at[dst_v.at[j]],
                                    add=True)

        slow = jnp.logical_and(c == slow_core, s >= 8)
        if slow_core == 1:
            o_fast = c * N_SUBCORES + s
        else:
            o_fast = jnp.where(c == 0, s, N_SUBCORES // 2 + s)

        with jax.named_scope("edges"):
            @pl.when(jnp.logical_not(slow))
            def _():
                run_edges(w_f, o_fast * w_f)

            @pl.when(slow)
            def _():
                run_edges(w_s, 24 * w_f + (s - 8) * w_s)

        plsc.subcore_barrier()

        with jax.named_scope("writeout"):
            pltpu.sync_copy(acc_sh.at[pl.ds(row0, rows_per_sub)],
                            part_hbm.at[c].at[pl.ds(row0, rows_per_sub)])
            if with_deg:
                pltpu.sync_copy(hist_v, degh_hbm.at[pl.ds(w * n_hist, n_hist)])

    return sc_agg


def _tc_layer1(x, part, degp, W1l, b1, W1r, W2l, W2r, b2):
    n, f_in = x.shape
    h_dim = W1l.shape[0]
    c_dim = W2l.shape[0]

    def body(x_ref, p_ref, d_ref, w1l_ref, b1_ref, w1r_ref, w2l_ref,
             w2r_ref, b2_ref, h2l_ref, h2r_ref):
        psum = p_ref[0, :n, :] + p_ref[1, :n, :]
        deg = lax.dot_general(d_ref[...], jnp.ones((N_WORKERS, 1), jnp.float32),
                              (((0,), (0,)), ((), ())),
                              preferred_element_type=jnp.float32)[:n, :]
        inv = 1.0 / jnp.maximum(deg, 1.0)
        agg = psum * inv
        dn = (((1,), (1,)), ((), ()))
        h = jnp.maximum(
            lax.dot_general(agg, w1l_ref[...], dn,
                            preferred_element_type=jnp.float32)
            + b1_ref[...]
            + lax.dot_general(x_ref[...], w1r_ref[...], dn,
                              preferred_element_type=jnp.float32),
            0.0)
        h2l_ref[...] = h
        h2r_ref[...] = lax.dot_general(h, w2r_ref[...], dn,
                                       preferred_element_type=jnp.float32) + b2_ref[...]

    return pl.pallas_call(
        body,
        out_shape=[jax.ShapeDtypeStruct((n, f_in), jnp.float32),
                   jax.ShapeDtypeStruct((n, c_dim), jnp.float32)],
    )(x, part, degp, W1l, b1.reshape(1, h_dim), W1r, W2l, W2r,
      b2.reshape(1, c_dim))


def _tc_layer2(part2, degp, W2l, h2r, gamma, beta):
    n, c_dim = h2r.shape

    def body(q_ref, d_ref, w2l_ref, h2r_ref, g_ref, b_ref, o_ref):
        qsum = q_ref[0, :n, :] + q_ref[1, :n, :]
        deg = lax.dot_general(d_ref[...], jnp.ones((N_WORKERS, 1), jnp.float32),
                              (((0,), (0,)), ((), ())),
                              preferred_element_type=jnp.float32)[:n, :]
        inv = 1.0 / jnp.maximum(deg, 1.0)
        agg = qsum * inv
        dn = (((1,), (1,)), ((), ()))
        pre = lax.dot_general(agg, w2l_ref[...], dn,
                              preferred_element_type=jnp.float32) + h2r_ref[...]
        mean = jnp.mean(pre, axis=0, keepdims=True)
        cent = pre - mean
        var = jnp.mean(cent * cent, axis=0, keepdims=True)
        o_ref[...] = cent * lax.rsqrt(var + 1e-5) * g_ref[...] + b_ref[...]

    return pl.pallas_call(
        body,
        out_shape=jax.ShapeDtypeStruct((n, c_dim), jnp.float32),
    )(part2, degp, W2l, h2r, gamma.reshape(1, c_dim), beta.reshape(1, c_dim))


def kernel(x, edge_index, W1l, b1, W1r, W2l, b2, W2r, gamma, beta):
    n, f_in = x.shape
    e = edge_index.shape[1]
    c_dim = W2l.shape[0]

    span = N_WORKERS * CHUNK * 8
    e_pad = ((e + span - 1) // span) * span
    n_chunks = e_pad // CHUNK
    n_acc = ((n + 1 + N_SUBCORES * CHUNK - 1)
             // (N_SUBCORES * CHUNK)) * (N_SUBCORES * CHUNK)

    pad = e_pad - e
    src = jnp.concatenate(
        [edge_index[0], jnp.zeros((pad,), jnp.int32)]).reshape(n_chunks, CHUNK)
    pad_dst = n + (jnp.arange(pad, dtype=jnp.int32) % 16)
    dst = jnp.concatenate(
        [edge_index[1], pad_dst]).reshape(n_chunks, CHUNK)

    zf1 = jnp.zeros((CHUNK, f_in), jnp.float32)
    n_hist = ((n + 1 + 15) // 16) * 16

    sc1 = _make_sc_agg(f_in, n_chunks, n_acc, with_deg=True, hist_rows=n_hist)
    part1, degh = sc1(x, src, dst, zf1)
    degp = degh.reshape(N_WORKERS, n_hist)

    h, h2r = _tc_layer1(x, part1, degp, W1l, b1, W1r, W2l, W2r, b2)

    sc2 = _make_sc_agg(f_in, n_chunks, n_acc, with_deg=False)
    (part2,) = sc2(h, src, dst, zf1)

    return _tc_layer2(part2, degp, W2l, h2r, gamma, beta)


# speedup vs baseline: 1.0250x; 1.0122x over previous
import dataclasses
import functools

import jax
import jax.numpy as jnp
from jax import lax
from jax.experimental import pallas as pl
from jax.experimental.pallas import tpu as pltpu
from jax.experimental.pallas import tpu_sc as plsc

N_SC_CORES = 2
N_SUBCORES = 16
N_WORKERS = N_SC_CORES * N_SUBCORES
CHUNK = 128


def _make_sc_agg(feat_dim, n_chunks, n_acc, with_deg, hist_rows=0,
                 slow_core=1, slow_blocks=4):
    idx_blk = 8
    total_blocks = n_chunks // idx_blk
    w_s = slow_blocks
    w_f = (total_blocks - 8 * w_s) // 24
    assert 24 * w_f + 8 * w_s == total_blocks, (w_f, w_s, total_blocks)
    rows_per_sub = n_acc // N_SUBCORES
    n_zero = rows_per_sub // CHUNK
    n_hist = hist_rows
    mesh = plsc.VectorSubcoreMesh(core_axis_name="c", subcore_axis_name="s")

    out_type = [jax.ShapeDtypeStruct((N_SC_CORES, n_acc, feat_dim), jnp.float32)]
    scratch = [
        pltpu.VMEM((idx_blk, CHUNK), jnp.int32),
        pltpu.VMEM((idx_blk, CHUNK), jnp.int32),
        pltpu.VMEM((CHUNK, feat_dim), jnp.float32),
        pltpu.VMEM((CHUNK, feat_dim), jnp.float32),
        pltpu.VMEM_SHARED((n_acc, feat_dim), jnp.float32),
        pltpu.SemaphoreType.DMA,
        pltpu.SemaphoreType.DMA,
    ]
    if with_deg:
        out_type.append(jax.ShapeDtypeStruct((N_WORKERS * n_hist,), jnp.float32))
        scratch.append(pltpu.VMEM((n_hist,), jnp.float32))

    cp = pltpu.CompilerParams()
    if "needs_layout_passes" in pltpu.CompilerParams.__dataclass_fields__:
        cp = dataclasses.replace(cp, needs_layout_passes=False)

    @functools.partial(pl.kernel, mesh=mesh, out_type=out_type,
                       scratch_types=scratch, compiler_params=cp)
    def sc_agg(*refs):
        if with_deg:
            (feat_hbm, srcs_hbm, dsts_hbm, zf_hbm,
             part_hbm, degh_hbm,
             src_v, dst_v, rows_a, rows_b, acc_sh, sem_a, sem_b, hist_v) = refs
        else:
            (feat_hbm, srcs_hbm, dsts_hbm, zf_hbm,
             part_hbm,
             src_v, dst_v, rows_a, rows_b, acc_sh, sem_a, sem_b) = refs
        c = lax.axis_index("c")
        s = lax.axis_index("s")
        w = c * N_SUBCORES + s
        row0 = s * rows_per_sub
        bufs = (rows_a, rows_b)
        sems = (sem_a, sem_b)

        with jax.named_scope("zero_acc"):
            pltpu.sync_copy(zf_hbm, rows_a)

            @pl.loop(0, n_zero)
            def _(k):
                pltpu.sync_copy(rows_a,
                                acc_sh.at[pl.ds(row0 + k * CHUNK, CHUNK)])

        if with_deg:
            zeros16 = jnp.zeros((16,), jnp.float32)

            @pl.loop(0, n_hist // 16)
            def _(i):
                hist_v[pl.ds(i * 16, 16)] = zeros16

        plsc.subcore_barrier()

        if with_deg:
            ones16 = jnp.full((16,), 1.0, jnp.float32)

        def run_edges(n_blocks, start_block):
            @pl.loop(0, n_blocks)
            def _(b):
                blk = (start_block + b) * idx_blk
                pltpu.sync_copy(srcs_hbm.at[pl.ds(blk, idx_blk)], src_v)
                pltpu.sync_copy(dsts_hbm.at[pl.ds(blk, idx_blk)], dst_v)

                copies = [None] * idx_blk
                copies[0] = pltpu.async_copy(
                    feat_hbm.at[src_v.at[0]], bufs[0], sems[0])
                for j in range(idx_blk):
                    if j + 1 < idx_blk:
                        copies[j + 1] = pltpu.async_copy(
                            feat_hbm.at[src_v.at[j + 1]],
                            bufs[(j + 1) % 2], sems[(j + 1) % 2])
                    if with_deg:
                        for k in range(CHUNK // 16):
                            idx = dst_v[j, pl.ds(k * 16, 16)]
                            plsc.addupdate_scatter(hist_v, [idx], ones16)
                    copies[j].wait()
                    pltpu.sync_copy(bufs[j % 2], acc_sh.at[dst_v.at[j]],
                                    add=True)

        slow = jnp.logical_and(c == slow_core, s >= 8)
        if slow_core == 1:
            o_fast = c * N_SUBCORES + s
        else:
            o_fast = jnp.where(c == 0, s, N_SUBCORES // 2 + s)

        with jax.named_scope("edges"):
            @pl.when(jnp.logical_not(slow))
            def _():
                run_edges(w_f, o_fast * w_f)

            @pl.when(slow)
            def _():
                run_edges(w_s, 24 * w_f + (s - 8) * w_s)

        plsc.subcore_barrier()

        with jax.named_scope("writeout"):
            pltpu.sync_copy(acc_sh.at[pl.ds(row0, rows_per_sub)],
                            part_hbm.at[c].at[pl.ds(row0, rows_per_sub)])
            if with_deg:
                pltpu.sync_copy(hist_v, degh_hbm.at[pl.ds(w * n_hist, n_hist)])

    return sc_agg


def _tc_layer1(x, part, degp, W1l, b1, W1r, W2l, W2r, b2):
    n, f_in = x.shape
    h_dim = W1l.shape[0]
    c_dim = W2l.shape[0]

    def body(x_ref, p_ref, d_ref, w1l_ref, b1_ref, w1r_ref, w2l_ref,
             w2r_ref, b2_ref, h2l_ref, h2r_ref):
        psum = p_ref[0, :n, :] + p_ref[1, :n, :]
        deg = lax.dot_general(d_ref[...], jnp.ones((N_WORKERS, 1), jnp.float32),
                              (((0,), (0,)), ((), ())),
                              preferred_element_type=jnp.float32)[:n, :]
        inv = 1.0 / jnp.maximum(deg, 1.0)
        agg = psum * inv
        dn = (((1,), (1,)), ((), ()))
        h = jnp.maximum(
            lax.dot_general(agg, w1l_ref[...], dn,
                            preferred_element_type=jnp.float32)
            + b1_ref[...]
            + lax.dot_general(x_ref[...], w1r_ref[...], dn,
                              preferred_element_type=jnp.float32),
            0.0)
        h2l_ref[...] = h
        h2r_ref[...] = lax.dot_general(h, w2r_ref[...], dn,
                                       preferred_element_type=jnp.float32) + b2_ref[...]

    return pl.pallas_call(
        body,
        out_shape=[jax.ShapeDtypeStruct((n, f_in), jnp.float32),
                   jax.ShapeDtypeStruct((n, c_dim), jnp.float32)],
    )(x, part, degp, W1l, b1.reshape(1, h_dim), W1r, W2l, W2r,
      b2.reshape(1, c_dim))


def _tc_layer2(part2, degp, W2l, h2r, gamma, beta):
    n, c_dim = h2r.shape

    def body(q_ref, d_ref, w2l_ref, h2r_ref, g_ref, b_ref, o_ref):
        qsum = q_ref[0, :n, :] + q_ref[1, :n, :]
        deg = lax.dot_general(d_ref[...], jnp.ones((N_WORKERS, 1), jnp.float32),
                              (((0,), (0,)), ((), ())),
                              preferred_element_type=jnp.float32)[:n, :]
        inv = 1.0 / jnp.maximum(deg, 1.0)
        agg = qsum * inv
        dn = (((1,), (1,)), ((), ()))
        pre = lax.dot_general(agg, w2l_ref[...], dn,
                              preferred_element_type=jnp.float32) + h2r_ref[...]
        mean = jnp.mean(pre, axis=0, keepdims=True)
        cent = pre - mean
        var = jnp.mean(cent * cent, axis=0, keepdims=True)
        o_ref[...] = cent * lax.rsqrt(var + 1e-5) * g_ref[...] + b_ref[...]

    return pl.pallas_call(
        body,
        out_shape=jax.ShapeDtypeStruct((n, c_dim), jnp.float32),
    )(part2, degp, W2l, h2r, gamma.reshape(1, c_dim), beta.reshape(1, c_dim))


def kernel(x, edge_index, W1l, b1, W1r, W2l, b2, W2r, gamma, beta):
    n, f_in = x.shape
    e = edge_index.shape[1]
    c_dim = W2l.shape[0]

    span = N_WORKERS * CHUNK * 8
    e_pad = ((e + span - 1) // span) * span
    n_chunks = e_pad // CHUNK
    n_acc = ((n + 1 + N_SUBCORES * CHUNK - 1)
             // (N_SUBCORES * CHUNK)) * (N_SUBCORES * CHUNK)

    pad = e_pad - e
    src = jnp.concatenate(
        [edge_index[0], jnp.zeros((pad,), jnp.int32)]).reshape(n_chunks, CHUNK)
    pad_dst = n + (jnp.arange(pad, dtype=jnp.int32) % 16)
    dst = jnp.concatenate(
        [edge_index[1], pad_dst]).reshape(n_chunks, CHUNK)

    zf1 = jnp.zeros((CHUNK, f_in), jnp.float32)
    n_hist = ((n + 1 + 15) // 16) * 16

    sc1 = _make_sc_agg(f_in, n_chunks, n_acc, with_deg=True, hist_rows=n_hist,
                       slow_core=0)
    part1, degh = sc1(x, src, dst, zf1)
    degp = degh.reshape(N_WORKERS, n_hist)

    h, h2r = _tc_layer1(x, part1, degp, W1l, b1, W1r, W2l, W2r, b2)

    sc2 = _make_sc_agg(f_in, n_chunks, n_acc, with_deg=False, slow_core=0)
    (part2,) = sc2(h, src, dst, zf1)

    return _tc_layer2(part2, degp, W2l, h2r, gamma, beta)


# speedup vs baseline: 2.7067x; 2.6407x over previous
import dataclasses
import functools

import jax
import jax.numpy as jnp
from jax import lax
from jax.experimental import pallas as pl
from jax.experimental.pallas import tpu as pltpu
from jax.experimental.pallas import tpu_sc as plsc

N_SC_CORES = 2
N_SUBCORES = 16
N_WORKERS = N_SC_CORES * N_SUBCORES
CHUNK = 128


def _make_sc_agg(feat_dim, n_chunks, n_acc, with_deg, hist_rows=0,
                 slow_core=1, slow_blocks=4):
    idx_blk = 8
    total_blocks = n_chunks // idx_blk
    w_s = slow_blocks
    w_f = (total_blocks - 8 * w_s) // 24
    assert 24 * w_f + 8 * w_s == total_blocks, (w_f, w_s, total_blocks)
    rows_per_sub = n_acc // N_SUBCORES
    n_zero = rows_per_sub // CHUNK
    n_hist = hist_rows
    mesh = plsc.VectorSubcoreMesh(core_axis_name="c", subcore_axis_name="s")

    out_type = [jax.ShapeDtypeStruct((N_SC_CORES, n_acc, feat_dim), jnp.float32)]
    scratch = [
        pltpu.VMEM((idx_blk, CHUNK), jnp.int32),
        pltpu.VMEM((idx_blk, CHUNK), jnp.int32),
        pltpu.VMEM((CHUNK, feat_dim), jnp.float32),
        pltpu.VMEM((CHUNK, feat_dim), jnp.float32),
        pltpu.VMEM_SHARED((n_acc, feat_dim), jnp.float32),
        pltpu.SemaphoreType.DMA,
        pltpu.SemaphoreType.DMA,
    ]
    if with_deg:
        out_type.append(jax.ShapeDtypeStruct((N_WORKERS * n_hist,), jnp.float32))
        scratch.append(pltpu.VMEM((n_hist,), jnp.float32))

    cp = pltpu.CompilerParams()
    if "needs_layout_passes" in pltpu.CompilerParams.__dataclass_fields__:
        cp = dataclasses.replace(cp, needs_layout_passes=False)

    @functools.partial(pl.kernel, mesh=mesh, out_type=out_type,
                       scratch_types=scratch, compiler_params=cp)
    def sc_agg(*refs):
        if with_deg:
            (feat_hbm, srcs_hbm, dsts_hbm, zf_hbm,
             part_hbm, degh_hbm,
             src_v, dst_v, rows_a, rows_b, acc_sh, sem_a, sem_b, hist_v) = refs
        else:
            (feat_hbm, srcs_hbm, dsts_hbm, zf_hbm,
             part_hbm,
             src_v, dst_v, rows_a, rows_b, acc_sh, sem_a, sem_b) = refs
        c = lax.axis_index("c")
        s = lax.axis_index("s")
        w = c * N_SUBCORES + s
        row0 = s * rows_per_sub
        bufs = (rows_a, rows_b)
        sems = (sem_a, sem_b)

        with jax.named_scope("zero_acc"):
            pltpu.sync_copy(zf_hbm, rows_a)

            @pl.loop(0, n_zero)
            def _(k):
                pltpu.sync_copy(rows_a,
                                acc_sh.at[pl.ds(row0 + k * CHUNK, CHUNK)])

        if with_deg:
            zeros16 = jnp.zeros((16,), jnp.float32)

            @pl.loop(0, n_hist // 16)
            def _(i):
                hist_v[pl.ds(i * 16, 16)] = zeros16

        plsc.subcore_barrier()

        if with_deg:
            ones16 = jnp.full((16,), 1.0, jnp.float32)

        def run_edges(n_blocks, start_block):
            @pl.loop(0, n_blocks)
            def _(b):
                blk = (start_block + b) * idx_blk
                pltpu.sync_copy(srcs_hbm.at[pl.ds(blk, idx_blk)], src_v)
                pltpu.sync_copy(dsts_hbm.at[pl.ds(blk, idx_blk)], dst_v)

                copies = [None] * idx_blk
                copies[0] = pltpu.async_copy(
                    feat_hbm.at[src_v.at[0]], bufs[0], sems[0])
                for j in range(idx_blk):
                    if j + 1 < idx_blk:
                        copies[j + 1] = pltpu.async_copy(
                            feat_hbm.at[src_v.at[j + 1]],
                            bufs[(j + 1) % 2], sems[(j + 1) % 2])
                    if with_deg:
                        for k in range(CHUNK // 16):
                            idx = dst_v[j, pl.ds(k * 16, 16)]
                            plsc.addupdate_scatter(hist_v, [idx], ones16)
                    copies[j].wait()
                    pltpu.sync_copy(bufs[j % 2], acc_sh.at[dst_v.at[j]],
                                    add=True)

        slow = jnp.logical_and(c == slow_core, s >= 8)
        if slow_core == 1:
            o_fast = c * N_SUBCORES + s
        else:
            o_fast = jnp.where(c == 0, s, N_SUBCORES // 2 + s)

        with jax.named_scope("edges"):
            @pl.when(jnp.logical_not(slow))
            def _():
                run_edges(w_f, o_fast * w_f)

            @pl.when(slow)
            def _():
                run_edges(w_s, 24 * w_f + (s - 8) * w_s)

        plsc.subcore_barrier()

        with jax.named_scope("writeout"):
            pltpu.sync_copy(acc_sh.at[pl.ds(row0, rows_per_sub)],
                            part_hbm.at[c].at[pl.ds(row0, rows_per_sub)])
            if with_deg:
                pltpu.sync_copy(hist_v, degh_hbm.at[pl.ds(w * n_hist, n_hist)])

    return sc_agg


def _tc_layer1(x, part, degp, W1l, b1, W1r, W2l, W2r, b2):
    n, f_in = x.shape
    h_dim = W1l.shape[0]
    c_dim = W2l.shape[0]

    def body(x_ref, p_ref, d_ref, w1l_ref, b1_ref, w1r_ref, w2l_ref,
             w2r_ref, b2_ref, h2l_ref, h2r_ref):
        psum = p_ref[0, :n, :] + p_ref[1, :n, :]
        deg = lax.dot_general(d_ref[...], jnp.ones((N_WORKERS, 1), jnp.float32),
                              (((0,), (0,)), ((), ())),
                              preferred_element_type=jnp.float32)[:n, :]
        inv = 1.0 / jnp.maximum(deg, 1.0)
        agg = psum * inv
        dn = (((1,), (1,)), ((), ()))
        h = jnp.maximum(
            lax.dot_general(agg, w1l_ref[...], dn,
                            preferred_element_type=jnp.float32)
            + b1_ref[...]
            + lax.dot_general(x_ref[...], w1r_ref[...], dn,
                              preferred_element_type=jnp.float32),
            0.0)
        h2l_ref[...] = h
        h2r_ref[...] = lax.dot_general(h, w2r_ref[...], dn,
                                       preferred_element_type=jnp.float32) + b2_ref[...]

    return pl.pallas_call(
        body,
        out_shape=[jax.ShapeDtypeStruct((n, f_in), jnp.float32),
                   jax.ShapeDtypeStruct((n, c_dim), jnp.float32)],
    )(x, part, degp, W1l, b1.reshape(1, h_dim), W1r, W2l, W2r,
      b2.reshape(1, c_dim))


def _tc_layer2(part2, degp, W2l, h2r, gamma, beta):
    n, c_dim = h2r.shape

    def body(q_ref, d_ref, w2l_ref, h2r_ref, g_ref, b_ref, o_ref):
        qsum = q_ref[0, :n, :] + q_ref[1, :n, :]
        deg = lax.dot_general(d_ref[...], jnp.ones((N_WORKERS, 1), jnp.float32),
                              (((0,), (0,)), ((), ())),
                              preferred_element_type=jnp.float32)[:n, :]
        inv = 1.0 / jnp.maximum(deg, 1.0)
        agg = qsum * inv
        dn = (((1,), (1,)), ((), ()))
        pre = lax.dot_general(agg, w2l_ref[...], dn,
                              preferred_element_type=jnp.float32) + h2r_ref[...]
        mean = jnp.mean(pre, axis=0, keepdims=True)
        cent = pre - mean
        var = jnp.mean(cent * cent, axis=0, keepdims=True)
        o_ref[...] = cent * lax.rsqrt(var + 1e-5) * g_ref[...] + b_ref[...]

    return pl.pallas_call(
        body,
        out_shape=jax.ShapeDtypeStruct((n, c_dim), jnp.float32),
    )(part2, degp, W2l, h2r, gamma.reshape(1, c_dim), beta.reshape(1, c_dim))


def kernel(x, edge_index, W1l, b1, W1r, W2l, b2, W2r, gamma, beta):
    n, f_in = x.shape
    e = edge_index.shape[1]
    c_dim = W2l.shape[0]

    span = N_WORKERS * CHUNK * 8
    e_pad = ((e + span - 1) // span) * span
    n_chunks = e_pad // CHUNK
    n_acc = ((n + 1 + N_SUBCORES * CHUNK - 1)
             // (N_SUBCORES * CHUNK)) * (N_SUBCORES * CHUNK)

    pad = e_pad - e
    pad_src = jnp.arange(pad, dtype=jnp.int32) % n
    src = jnp.concatenate(
        [edge_index[0], pad_src]).reshape(n_chunks, CHUNK)
    pad_dst = n + (jnp.arange(pad, dtype=jnp.int32) % 16)
    dst = jnp.concatenate(
        [edge_index[1], pad_dst]).reshape(n_chunks, CHUNK)

    zf1 = jnp.zeros((CHUNK, f_in), jnp.float32)
    n_hist = ((n + 1 + 15) // 16) * 16

    sc1 = _make_sc_agg(f_in, n_chunks, n_acc, with_deg=True, hist_rows=n_hist,
                       slow_blocks=10)
    part1, degh = sc1(x, src, dst, zf1)
    degp = degh.reshape(N_WORKERS, n_hist)

    h, h2r = _tc_layer1(x, part1, degp, W1l, b1, W1r, W2l, W2r, b2)

    sc2 = _make_sc_agg(f_in, n_chunks, n_acc, with_deg=False, slow_blocks=10)
    (part2,) = sc2(h, src, dst, zf1)

    return _tc_layer2(part2, degp, W2l, h2r, gamma, beta)


# speedup vs baseline: 2.7082x; 1.0006x over previous
import dataclasses
import functools

import jax
import jax.numpy as jnp
from jax import lax
from jax.experimental import pallas as pl
from jax.experimental.pallas import tpu as pltpu
from jax.experimental.pallas import tpu_sc as plsc

N_SC_CORES = 2
N_SUBCORES = 16
N_WORKERS = N_SC_CORES * N_SUBCORES
CHUNK = 128


def _make_sc_agg(feat_dim, n_chunks, n_acc, with_deg, hist_rows=0):
    idx_blk = 8
    blocks_per_worker = n_chunks // (idx_blk * N_WORKERS)
    rows_per_sub = n_acc // N_SUBCORES
    n_zero = rows_per_sub // CHUNK
    n_hist = hist_rows
    mesh = plsc.VectorSubcoreMesh(core_axis_name="c", subcore_axis_name="s")

    out_type = [jax.ShapeDtypeStruct((N_SC_CORES, n_acc, feat_dim), jnp.float32)]
    scratch = [
        pltpu.VMEM((idx_blk, CHUNK), jnp.int32),
        pltpu.VMEM((idx_blk, CHUNK), jnp.int32),
        pltpu.VMEM((CHUNK, feat_dim), jnp.float32),
        pltpu.VMEM((CHUNK, feat_dim), jnp.float32),
        pltpu.VMEM_SHARED((n_acc, feat_dim), jnp.float32),
        pltpu.SemaphoreType.DMA,
        pltpu.SemaphoreType.DMA,
    ]
    if with_deg:
        out_type.append(jax.ShapeDtypeStruct((N_WORKERS * n_hist,), jnp.float32))
        scratch.append(pltpu.VMEM((n_hist,), jnp.float32))

    cp = pltpu.CompilerParams()
    if "needs_layout_passes" in pltpu.CompilerParams.__dataclass_fields__:
        cp = dataclasses.replace(cp, needs_layout_passes=False)

    @functools.partial(pl.kernel, mesh=mesh, out_type=out_type,
                       scratch_types=scratch, compiler_params=cp)
    def sc_agg(*refs):
        if with_deg:
            (feat_hbm, srcs_hbm, dsts_hbm, zf_hbm,
             part_hbm, degh_hbm,
             src_v, dst_v, rows_a, rows_b, acc_sh, sem_a, sem_b, hist_v) = refs
        else:
            (feat_hbm, srcs_hbm, dsts_hbm, zf_hbm,
             part_hbm,
             src_v, dst_v, rows_a, rows_b, acc_sh, sem_a, sem_b) = refs
        c = lax.axis_index("c")
        s = lax.axis_index("s")
        w = c * N_SUBCORES + s
        row0 = s * rows_per_sub
        bufs = (rows_a, rows_b)
        sems = (sem_a, sem_b)

        with jax.named_scope("zero_acc"):
            pltpu.sync_copy(zf_hbm, rows_a)

            @pl.loop(0, n_zero)
            def _(k):
                pltpu.sync_copy(rows_a,
                                acc_sh.at[pl.ds(row0 + k * CHUNK, CHUNK)])

        if with_deg:
            zeros16 = jnp.zeros((16,), jnp.float32)

            @pl.loop(0, n_hist // 16)
            def _(i):
                hist_v[pl.ds(i * 16, 16)] = zeros16

        plsc.subcore_barrier()

        if with_deg:
            ones16 = jnp.full((16,), 1.0, jnp.float32)

        def run_edges(n_blocks, start_block):
            @pl.loop(0, n_blocks)
            def _(b):
                blk = (start_block + b) * idx_blk
                pltpu.sync_copy(srcs_hbm.at[pl.ds(blk, idx_blk)], src_v)
                pltpu.sync_copy(dsts_hbm.at[pl.ds(blk, idx_blk)], dst_v)

                copies = [None] * idx_blk
                copies[0] = pltpu.async_copy(
                    feat_hbm.at[src_v.at[0]], bufs[0], sems[0])
                for j in range(idx_blk):
                    if j + 1 < idx_blk:
                        copies[j + 1] = pltpu.async_copy(
                            feat_hbm.at[src_v.at[j + 1]],
                            bufs[(j + 1) % 2], sems[(j + 1) % 2])
                    if with_deg:
                        for k in range(CHUNK // 16):
                            idx = dst_v[j, pl.ds(k * 16, 16)]
                            plsc.addupdate_scatter(hist_v, [idx], ones16)
                    copies[j].wait()
                    pltpu.sync_copy(bufs[j % 2], acc_sh.at[dst_v.at[j]],
                                    add=True)

        with jax.named_scope("edges"):
            run_edges(blocks_per_worker, w * blocks_per_worker)

        plsc.subcore_barrier()

        with jax.named_scope("writeout"):
            pltpu.sync_copy(acc_sh.at[pl.ds(row0, rows_per_sub)],
                            part_hbm.at[c].at[pl.ds(row0, rows_per_sub)])
            if with_deg:
                pltpu.sync_copy(hist_v, degh_hbm.at[pl.ds(w * n_hist, n_hist)])

    return sc_agg


def _tc_layer1(x, part, degp, W1l, b1, W1r, W2l, W2r, b2):
    n, f_in = x.shape
    h_dim = W1l.shape[0]
    c_dim = W2l.shape[0]

    def body(x_ref, p_ref, d_ref, w1l_ref, b1_ref, w1r_ref, w2l_ref,
             w2r_ref, b2_ref, h2l_ref, h2r_ref):
        psum = p_ref[0, :n, :] + p_ref[1, :n, :]
        deg = lax.dot_general(d_ref[...], jnp.ones((N_WORKERS, 1), jnp.float32),
                              (((0,), (0,)), ((), ())),
                              preferred_element_type=jnp.float32)[:n, :]
        inv = 1.0 / jnp.maximum(deg, 1.0)
        agg = psum * inv
        dn = (((1,), (1,)), ((), ()))
        h = jnp.maximum(
            lax.dot_general(agg, w1l_ref[...], dn,
                            preferred_element_type=jnp.float32)
            + b1_ref[...]
            + lax.dot_general(x_ref[...], w1r_ref[...], dn,
                              preferred_element_type=jnp.float32),
            0.0)
        h2l_ref[...] = h
        h2r_ref[...] = lax.dot_general(h, w2r_ref[...], dn,
                                       preferred_element_type=jnp.float32) + b2_ref[...]

    return pl.pallas_call(
        body,
        out_shape=[jax.ShapeDtypeStruct((n, f_in), jnp.float32),
                   jax.ShapeDtypeStruct((n, c_dim), jnp.float32)],
    )(x, part, degp, W1l, b1.reshape(1, h_dim), W1r, W2l, W2r,
      b2.reshape(1, c_dim))


def _tc_layer2(part2, degp, W2l, h2r, gamma, beta):
    n, c_dim = h2r.shape

    def body(q_ref, d_ref, w2l_ref, h2r_ref, g_ref, b_ref, o_ref):
        qsum = q_ref[0, :n, :] + q_ref[1, :n, :]
        deg = lax.dot_general(d_ref[...], jnp.ones((N_WORKERS, 1), jnp.float32),
                              (((0,), (0,)), ((), ())),
                              preferred_element_type=jnp.float32)[:n, :]
        inv = 1.0 / jnp.maximum(deg, 1.0)
        agg = qsum * inv
        dn = (((1,), (1,)), ((), ()))
        pre = lax.dot_general(agg, w2l_ref[...], dn,
                              preferred_element_type=jnp.float32) + h2r_ref[...]
        mean = jnp.mean(pre, axis=0, keepdims=True)
        cent = pre - mean
        var = jnp.mean(cent * cent, axis=0, keepdims=True)
        o_ref[...] = cent * lax.rsqrt(var + 1e-5) * g_ref[...] + b_ref[...]

    return pl.pallas_call(
        body,
        out_shape=jax.ShapeDtypeStruct((n, c_dim), jnp.float32),
    )(part2, degp, W2l, h2r, gamma.reshape(1, c_dim), beta.reshape(1, c_dim))


def kernel(x, edge_index, W1l, b1, W1r, W2l, b2, W2r, gamma, beta):
    n, f_in = x.shape
    e = edge_index.shape[1]
    c_dim = W2l.shape[0]

    span = N_WORKERS * CHUNK * 8
    e_pad = ((e + span - 1) // span) * span
    n_chunks = e_pad // CHUNK
    n_acc = ((n + 1 + N_SUBCORES * CHUNK - 1)
             // (N_SUBCORES * CHUNK)) * (N_SUBCORES * CHUNK)

    pad = e_pad - e
    pad_src = jnp.arange(pad, dtype=jnp.int32) % n
    src = jnp.concatenate(
        [edge_index[0], pad_src]).reshape(n_chunks, CHUNK)
    pad_dst = n + (jnp.arange(pad, dtype=jnp.int32) % 16)
    dst = jnp.concatenate(
        [edge_index[1], pad_dst]).reshape(n_chunks, CHUNK)

    zf1 = jnp.zeros((CHUNK, f_in), jnp.float32)
    n_hist = ((n + 1 + 15) // 16) * 16

    sc1 = _make_sc_agg(f_in, n_chunks, n_acc, with_deg=True, hist_rows=n_hist)
    part1, degh = sc1(x, src, dst, zf1)
    degp = degh.reshape(N_WORKERS, n_hist)

    h, h2r = _tc_layer1(x, part1, degp, W1l, b1, W1r, W2l, W2r, b2)

    sc2 = _make_sc_agg(f_in, n_chunks, n_acc, with_deg=False)
    (part2,) = sc2(h, src, dst, zf1)

    return _tc_layer2(part2, degp, W2l, h2r, gamma, beta)
